# baseline (speedup 1.0000x reference)
import jax
import jax.numpy as jnp
from jax import lax
from jax.experimental import pallas as pl
from jax.experimental.pallas import tpu as pltpu, tpu_sc as plsc

_NW = 32
_TOT = 96 * 512 * 512
_COPY = 80 * 512 * 512
_MASKN = 16 * 512 * 512
_CPW = _COPY // _NW
_MPW = _MASKN // _NW
_VCH = 32768
_NCH = _MPW // _VCH


def _sc_body(x_hbm, m_hbm, o_hbm, buf, sem):
    c = lax.axis_index("c")
    s = lax.axis_index("s")
    wid = s * 2 + c
    base = wid * _CPW
    cp = pltpu.make_async_copy(
        x_hbm.at[pl.ds(base, _CPW)], o_hbm.at[pl.ds(base, _CPW)], sem
    )
    cp.start()
    mbase = wid * _MPW
    for ch in range(_NCH):
        off = mbase + ch * _VCH
        pltpu.sync_copy(m_hbm.at[pl.ds(off, _VCH)], buf)

        def body(i, carry):
            sl = pl.ds(i * 16, 16)
            v = buf[sl]
            buf[sl] = 1.0 / (1.0 + jnp.exp(-v))
            return carry

        lax.fori_loop(0, _VCH // 16, body, 0, unroll=8)
        pltpu.sync_copy(buf, o_hbm.at[pl.ds(_COPY + off, _VCH)])
    cp.wait()


def kernel(x, mask):
    xf = x.reshape(_TOT)
    mf = mask.reshape(_MASKN)
    mesh = plsc.VectorSubcoreMesh(core_axis_name="c", subcore_axis_name="s")
    out = pl.kernel(
        _sc_body,
        mesh=mesh,
        out_type=jax.ShapeDtypeStruct((_TOT,), jnp.float32),
        scratch_types=[
            pltpu.VMEM((_VCH,), jnp.float32),
            pltpu.SemaphoreType.DMA,
        ],
    )(xf, mf)
    return out.reshape(x.shape)
